# Initial kernel scaffold; baseline (speedup 1.0000x reference)
import jax
import jax.numpy as jnp
from jax.experimental import pallas as pl


def kernel(sent, mask, word_table, mask_table):
    raise NotImplementedError("write your pallas kernel here")


# speedup vs baseline: 1.3925x; 1.3925x over previous
import functools

import jax
import jax.numpy as jnp
from jax import lax
from jax.experimental import pallas as pl
from jax.experimental.pallas import tpu as pltpu
from jax.experimental.pallas import tpu_sc as plsc

VOCAB = 1000000
EMB_DIM = 64
MASK_DIM = 50
B = 4096
L = 200
ROWS = B * L
OUT_DIM = EMB_DIM + MASK_DIM

NC, NS = 2, 16
NW = NC * NS
ROWS_W = ROWS // NW
CHUNK = 512
NCHUNK = ROWS_W // CHUNK
G = 128
NG = CHUNK // G


def _worker_id():
    return lax.axis_index("s") * NC + lax.axis_index("c")


def _gather_idx(sidx, g):
    return sidx.at[g]


def _body(sent_ref, mask_ref, word_ref, mt_ref, out_ref,
          sidx, midx, wbuf, obuf, mt_v, sem, osem):
    wid = _worker_id()

    pltpu.sync_copy(mt_ref, mt_v)
    b0 = [lax.bitcast_convert_type(mt_v[pl.ds(o, 16)], jnp.int32)
          for o in (0, 16, 32, 34)]
    b1 = [lax.bitcast_convert_type(mt_v[pl.ds(o, 16)], jnp.int32)
          for o in (64, 80, 96, 98)]
    bx = [a ^ b for a, b in zip(b0, b1)]

    def chunk_body(c, _):
        rowbase = wid * ROWS_W + c * CHUNK
        pltpu.sync_copy(sent_ref.at[pl.ds(wid * (ROWS_W // G) + c * NG, NG)],
                        sidx)
        pltpu.sync_copy(mask_ref.at[pl.ds(rowbase, CHUNK)],
                        midx.at[pl.ds(0, CHUNK)])
        copies = [
            pltpu.async_copy(word_ref.at[_gather_idx(sidx, g)],
                             wbuf.at[pl.ds(g * G, G)], sem)
            for g in range(NG)
        ]
        for cp in copies:
            cp.wait()

        def row_body(r, _):
            ob = r * OUT_DIM
            for k in range(4):
                obuf[pl.ds(ob + k * 16, 16)] = wbuf[r, pl.ds(k * 16, 16)]
            m = midx[pl.ds(r, 16)][0]
            mv = jnp.full((16,), -m, dtype=jnp.int32)
            sel = [lax.bitcast_convert_type(b ^ (x & mv), jnp.float32)
                   for b, x in zip(b0, bx)]
            obuf[pl.ds(ob + 64, 16)] = sel[0]
            obuf[pl.ds(ob + 80, 16)] = sel[1]
            obuf[pl.ds(ob + 96, 16)] = sel[2]
            obuf[pl.ds(ob + 98, 16)] = sel[3]
            return 0

        lax.fori_loop(0, CHUNK, row_body, 0)

        pltpu.async_copy(obuf, out_ref.at[pl.ds(rowbase * OUT_DIM,
                                                CHUNK * OUT_DIM)], osem).wait()
        return 0

    lax.fori_loop(0, NCHUNK, chunk_body, 0)


@jax.jit
def _run(sent2d, mask1d, word_table, mt_pad):
    mesh = plsc.VectorSubcoreMesh(core_axis_name="c", subcore_axis_name="s")
    k = pl.kernel(
        _body,
        out_type=jax.ShapeDtypeStruct((ROWS * OUT_DIM,), jnp.float32),
        mesh=mesh,
        compiler_params=pltpu.CompilerParams(use_tc_tiling_on_sc=False,
                                             needs_layout_passes=False),
        scratch_types=[
            pltpu.VMEM((NG, G), jnp.int32),
            pltpu.VMEM((CHUNK + 16,), jnp.int32),
            pltpu.VMEM((CHUNK, EMB_DIM), jnp.float32),
            pltpu.VMEM((CHUNK * OUT_DIM,), jnp.float32),
            pltpu.VMEM((128,), jnp.float32),
            pltpu.SemaphoreType.DMA,
            pltpu.SemaphoreType.DMA,
        ],
    )
    return k(sent2d, mask1d, word_table, mt_pad)


def kernel(sent, mask, word_table, mask_table):
    sent2d = sent.reshape(ROWS // G, G)
    mask1d = mask.reshape(ROWS)
    mt_pad = jnp.zeros((128,), jnp.float32)
    mt_pad = mt_pad.at[0:MASK_DIM].set(mask_table[0])
    mt_pad = mt_pad.at[64:64 + MASK_DIM].set(mask_table[1])
    out = _run(sent2d, mask1d, word_table, mt_pad)
    return out.reshape(B, L, OUT_DIM)


# speedup vs baseline: 1.8110x; 1.3005x over previous
import functools

import jax
import jax.numpy as jnp
from jax import lax
from jax.experimental import pallas as pl
from jax.experimental.pallas import tpu as pltpu
from jax.experimental.pallas import tpu_sc as plsc

VOCAB = 1000000
EMB_DIM = 64
MASK_DIM = 50
B = 4096
L = 200
ROWS = B * L
OUT_DIM = EMB_DIM + MASK_DIM

NC, NS = 2, 16
NW = NC * NS
ROWS_W = ROWS // NW
CHUNK = 256
NCHUNK = ROWS_W // CHUNK
G = 128
NG = CHUNK // G


def _worker_id():
    return lax.axis_index("s") * NC + lax.axis_index("c")


def _gather_idx(sidx, b, g):
    return sidx.at[b, g]


def _body(sent_ref, mask_ref, word_ref, mt_ref, out_ref,
          sidx, midx, wbuf, obuf, mt_v, gsem0, gsem1, osem0, osem1):
    wid = _worker_id()
    gsem = (gsem0, gsem1)
    osem = (osem0, osem1)

    pltpu.sync_copy(mt_ref, mt_v)
    b0 = [lax.bitcast_convert_type(mt_v[pl.ds(o, 16)], jnp.int32)
          for o in (0, 16, 32, 34)]
    b1 = [lax.bitcast_convert_type(mt_v[pl.ds(o, 16)], jnp.int32)
          for o in (64, 80, 96, 98)]
    bx = [a ^ b for a, b in zip(b0, b1)]

    def fire_gathers(c, b):
        pltpu.sync_copy(sent_ref.at[pl.ds(wid * (ROWS_W // G) + c * NG, NG)],
                        sidx.at[b])
        for g in range(NG):
            pltpu.async_copy(word_ref.at[_gather_idx(sidx, b, g)],
                             wbuf.at[b, pl.ds(g * G, G)], gsem[b])

    def do_chunk(c, b, first, last):
        rowbase = wid * ROWS_W + c * CHUNK

        if not last:
            fire_gathers(c + 1, 1 - b)

        pltpu.make_async_copy(word_ref.at[pl.ds(0, CHUNK)], wbuf.at[b],
                              gsem[b]).wait()
        pltpu.sync_copy(mask_ref.at[pl.ds(rowbase, CHUNK)],
                        midx.at[pl.ds(0, CHUNK)])

        @pl.when(jnp.logical_not(first))
        def _():
            pltpu.make_async_copy(obuf.at[b], out_ref.at[pl.ds(0, CHUNK)],
                                  osem[b]).wait()

        ob = obuf.at[b]
        wb = wbuf.at[b]

        def row_body(r, _):
            for kk in range(4):
                ob[r, pl.ds(kk * 16, 16)] = wb[r, pl.ds(kk * 16, 16)]
            mvals = plsc.load_gather(midx, [jnp.full((16,), r, jnp.int32)])
            mv = -mvals
            sel = [lax.bitcast_convert_type(p ^ (x & mv), jnp.float32)
                   for p, x in zip(b0, bx)]
            ob[r, pl.ds(64, 16)] = sel[0]
            ob[r, pl.ds(80, 16)] = sel[1]
            ob[r, pl.ds(96, 16)] = sel[2]
            ob[r, pl.ds(98, 16)] = sel[3]
            return 0

        lax.fori_loop(0, CHUNK, row_body, 0)

        pltpu.async_copy(ob, out_ref.at[pl.ds(rowbase, CHUNK)], osem[b])

    fire_gathers(0, 0)

    def pair_body(k, _):
        do_chunk(2 * k, 0, k == 0, False)

        @pl.when(k < NCHUNK // 2 - 1)
        def _():
            do_chunk(2 * k + 1, 1, k == 0, False)
        return 0

    lax.fori_loop(0, NCHUNK // 2, pair_body, 0)
    do_chunk(NCHUNK - 1, 1, False, True)

    for b in range(2):
        pltpu.make_async_copy(obuf.at[b], out_ref.at[pl.ds(0, CHUNK)],
                              osem[b]).wait()


@jax.jit
def _run(sent2d, mask1d, word_table, mt_pad):
    mesh = plsc.VectorSubcoreMesh(core_axis_name="c", subcore_axis_name="s")
    k = pl.kernel(
        _body,
        out_type=jax.ShapeDtypeStruct((ROWS, OUT_DIM), jnp.float32),
        mesh=mesh,
        compiler_params=pltpu.CompilerParams(use_tc_tiling_on_sc=False,
                                             needs_layout_passes=False),
        scratch_types=[
            pltpu.VMEM((2, NG, G), jnp.int32),
            pltpu.VMEM((CHUNK + 16,), jnp.int32),
            pltpu.VMEM((2, CHUNK, EMB_DIM), jnp.float32),
            pltpu.VMEM((2, CHUNK, OUT_DIM), jnp.float32),
            pltpu.VMEM((128,), jnp.float32),
            pltpu.SemaphoreType.DMA,
            pltpu.SemaphoreType.DMA,
            pltpu.SemaphoreType.DMA,
            pltpu.SemaphoreType.DMA,
        ],
    )
    return k(sent2d, mask1d, word_table, mt_pad)


def kernel(sent, mask, word_table, mask_table):
    sent2d = sent.reshape(ROWS // G, G)
    mask1d = mask.reshape(ROWS)
    mt_pad = jnp.zeros((128,), jnp.float32)
    mt_pad = mt_pad.at[0:MASK_DIM].set(mask_table[0])
    mt_pad = mt_pad.at[64:64 + MASK_DIM].set(mask_table[1])
    out = _run(sent2d, mask1d, word_table, mt_pad)
    return out.reshape(B, L, OUT_DIM)
